# baseline (speedup 1.0000x reference)
import functools

import jax
import jax.numpy as jnp
from jax.experimental import pallas as pl

NUM_EXPERTS = 8
TOP_K = 2


def _router_kernel(x_ref, gw_ref, comb_ref):
    x = x_ref[...]
    gw = gw_ref[...]
    logits = jax.lax.dot_general(
        x, gw, (((1,), (1,)), ((), ())),
        preferred_element_type=jnp.float32,
        precision=jax.lax.Precision.DEFAULT)
    p = jax.nn.softmax(logits, axis=-1)
    e_dim = p.shape[-1]
    iota = jax.lax.broadcasted_iota(jnp.int32, p.shape, 1)
    m1 = jnp.max(p, axis=-1, keepdims=True)
    i1 = jnp.min(jnp.where(p == m1, iota, e_dim), axis=-1, keepdims=True)
    mask1 = iota == i1
    pm = jnp.where(mask1, -jnp.inf, p)
    m2 = jnp.max(pm, axis=-1, keepdims=True)
    i2 = jnp.min(jnp.where(pm == m2, iota, e_dim), axis=-1, keepdims=True)
    mask2 = iota == i2
    sel = jnp.logical_or(mask1, mask2)
    comb_ref[...] = jnp.where(sel, p, 0.0) / (m1 + m2)


def _moe_kernel(x_ref, comb_ref, gw_ref, uw_ref, dw_ref, out_ref):
    e = pl.program_id(0)
    tb = pl.program_id(1)
    tblk = x_ref.shape[0]
    x = x_ref[...]
    g = jax.lax.dot_general(
        x, gw_ref[0], (((1,), (1,)), ((), ())),
        preferred_element_type=jnp.float32,
        precision=jax.lax.Precision.HIGHEST)
    u = jax.lax.dot_general(
        x, uw_ref[0], (((1,), (1,)), ((), ())),
        preferred_element_type=jnp.float32,
        precision=jax.lax.Precision.HIGHEST)
    h = (g * jax.lax.logistic(g)) * u
    y = jax.lax.dot_general(
        h, dw_ref[0], (((1,), (1,)), ((), ())),
        preferred_element_type=jnp.float32,
        precision=jax.lax.Precision.HIGHEST)
    comb = comb_ref[...]
    eio = jax.lax.broadcasted_iota(jnp.int32, comb.shape, 1)
    c = jnp.sum(jnp.where(eio == e, comb, 0.0), axis=1, keepdims=True)
    contrib = y * c
    rows = pl.ds(tb * tblk, tblk)

    @pl.when(e == 0)
    def _():
        out_ref[rows, :] = contrib

    @pl.when(e > 0)
    def _():
        out_ref[rows, :] += contrib


@functools.partial(jax.jit, static_argnames=("interpret",))
def kernel(hidden_states, gate_w, gate_proj, up_proj, down_proj,
           interpret=False):
    b, s, h = hidden_states.shape
    x = hidden_states.reshape(-1, h)
    t = x.shape[0]
    f = gate_proj.shape[1]

    comb = pl.pallas_call(
        _router_kernel,
        out_shape=jax.ShapeDtypeStruct((t, NUM_EXPERTS), jnp.float32),
        interpret=interpret,
    )(x, gate_w)

    tblk = 512
    out = pl.pallas_call(
        _moe_kernel,
        grid=(NUM_EXPERTS, t // tblk),
        in_specs=[
            pl.BlockSpec((tblk, h), lambda e, tb: (tb, 0)),
            pl.BlockSpec((tblk, NUM_EXPERTS), lambda e, tb: (tb, 0)),
            pl.BlockSpec((1, f, h), lambda e, tb: (e, 0, 0)),
            pl.BlockSpec((1, f, h), lambda e, tb: (e, 0, 0)),
            pl.BlockSpec((1, h, f), lambda e, tb: (e, 0, 0)),
        ],
        out_specs=pl.BlockSpec((t, h), lambda e, tb: (0, 0)),
        out_shape=jax.ShapeDtypeStruct((t, h), jnp.float32),
        interpret=interpret,
    )(x, comb, gate_proj, up_proj, down_proj)
    return out.reshape(b, s, h)


# speedup vs baseline: 4.1086x; 4.1086x over previous
import functools

import jax
import jax.numpy as jnp
from jax.experimental import pallas as pl

NUM_EXPERTS = 8
TOP_K = 2


def _router_kernel(x_ref, gw_ref, comb_ref):
    x = x_ref[...]
    gw = gw_ref[...]
    logits = jax.lax.dot_general(
        x, gw, (((1,), (1,)), ((), ())),
        preferred_element_type=jnp.float32,
        precision=jax.lax.Precision.DEFAULT)
    p = jax.nn.softmax(logits, axis=-1)
    e_dim = p.shape[-1]
    iota = jax.lax.broadcasted_iota(jnp.int32, p.shape, 1)
    m1 = jnp.max(p, axis=-1, keepdims=True)
    i1 = jnp.min(jnp.where(p == m1, iota, e_dim), axis=-1, keepdims=True)
    mask1 = iota == i1
    pm = jnp.where(mask1, -jnp.inf, p)
    m2 = jnp.max(pm, axis=-1, keepdims=True)
    i2 = jnp.min(jnp.where(pm == m2, iota, e_dim), axis=-1, keepdims=True)
    mask2 = iota == i2
    sel = jnp.logical_or(mask1, mask2)
    comb_ref[...] = jnp.where(sel, p, 0.0) / (m1 + m2)


def _moe_kernel(x_ref, comb_ref, gw_ref, uw_ref, dw_ref, out_ref):
    e = pl.program_id(0)
    tb = pl.program_id(1)
    tblk = x_ref.shape[0]
    x = x_ref[...]
    g = jax.lax.dot_general(
        x, gw_ref[0], (((1,), (1,)), ((), ())),
        preferred_element_type=jnp.float32,
        precision=jax.lax.Precision.DEFAULT)
    u = jax.lax.dot_general(
        x, uw_ref[0], (((1,), (1,)), ((), ())),
        preferred_element_type=jnp.float32,
        precision=jax.lax.Precision.DEFAULT)
    h = (g * jax.lax.logistic(g)) * u
    y = jax.lax.dot_general(
        h, dw_ref[0], (((1,), (1,)), ((), ())),
        preferred_element_type=jnp.float32,
        precision=jax.lax.Precision.DEFAULT)
    comb = comb_ref[...]
    eio = jax.lax.broadcasted_iota(jnp.int32, comb.shape, 1)
    c = jnp.sum(jnp.where(eio == e, comb, 0.0), axis=1, keepdims=True)
    contrib = y * c
    rows = pl.ds(tb * tblk, tblk)

    @pl.when(e == 0)
    def _():
        out_ref[rows, :] = contrib

    @pl.when(e > 0)
    def _():
        out_ref[rows, :] += contrib


@functools.partial(jax.jit, static_argnames=("interpret",))
def kernel(hidden_states, gate_w, gate_proj, up_proj, down_proj,
           interpret=False):
    b, s, h = hidden_states.shape
    x = hidden_states.reshape(-1, h)
    t = x.shape[0]
    f = gate_proj.shape[1]

    comb = pl.pallas_call(
        _router_kernel,
        out_shape=jax.ShapeDtypeStruct((t, NUM_EXPERTS), jnp.float32),
        interpret=interpret,
    )(x, gate_w)

    tblk = 512
    out = pl.pallas_call(
        _moe_kernel,
        grid=(NUM_EXPERTS, t // tblk),
        in_specs=[
            pl.BlockSpec((tblk, h), lambda e, tb: (tb, 0)),
            pl.BlockSpec((tblk, NUM_EXPERTS), lambda e, tb: (tb, 0)),
            pl.BlockSpec((1, f, h), lambda e, tb: (e, 0, 0)),
            pl.BlockSpec((1, f, h), lambda e, tb: (e, 0, 0)),
            pl.BlockSpec((1, h, f), lambda e, tb: (e, 0, 0)),
        ],
        out_specs=pl.BlockSpec((t, h), lambda e, tb: (0, 0)),
        out_shape=jax.ShapeDtypeStruct((t, h), jnp.float32),
        interpret=interpret,
    )(x, comb, gate_proj, up_proj, down_proj)
    return out.reshape(b, s, h)
